# Initial kernel scaffold; baseline (speedup 1.0000x reference)
import jax
import jax.numpy as jnp
from jax.experimental import pallas as pl


def kernel(agents, a_f, a_sim, ay, b_f, b_sim):
    raise NotImplementedError("write your pallas kernel here")


# speedup vs baseline: 2.1138x; 2.1138x over previous
import functools

import jax
import jax.numpy as jnp
from jax.experimental import pallas as pl
from jax.experimental.pallas import tpu as pltpu

_MARGIN = 1.0
_SIM_MARGIN = 1.0 - _MARGIN / 2.0


def _terms(f, agents, sim, lab):
    f2 = jnp.sum(f * f, axis=1, keepdims=True)
    a2 = jnp.sum(agents * agents, axis=1)[None, :]
    xdot = jax.lax.dot_general(
        f, agents, (((1,), (1,)), ((), ())), preferred_element_type=jnp.float32
    )
    sdist = (f2 + a2) - 2.0 * xdot
    neg = jnp.maximum(0.0, _MARGIN - sdist)
    simmask = sim > _SIM_MARGIN
    if lab is not None:
        cols = jax.lax.broadcasted_iota(jnp.int32, sdist.shape, 1)
        labmask = cols == lab
        mask = simmask & jnp.logical_not(labmask)
        pos = jnp.sum(jnp.where(labmask, sdist, 0.0))
    else:
        mask = simmask
        pos = 0.0
    cnt = jnp.sum(mask.astype(jnp.float32), axis=1)
    msum = jnp.sum(jnp.where(mask, neg, 0.0), axis=1)
    has = cnt > 0.0
    mean_neg = jnp.where(has, msum / jnp.maximum(cnt, 1.0), 0.0)
    num = pos + jnp.sum(mean_neg)
    den = jnp.sum(has.astype(jnp.float32))
    if lab is not None:
        den = den + float(f.shape[0])
    return num, den


def _body(nsteps, ag_ref, af_ref, asim_ref, ay_ref, bf_ref, bsim_ref,
          out_ref, acc_ref):
    i = pl.program_id(0)

    @pl.when(i == 0)
    def _init():
        acc_ref[0] = 0.0
        acc_ref[1] = 0.0

    agents = ag_ref[...]
    num_a, den_a = _terms(af_ref[...], agents, asim_ref[...], ay_ref[...])
    num_b, den_b = _terms(bf_ref[...], agents, bsim_ref[...], None)
    acc_ref[0] += num_a + num_b
    acc_ref[1] += den_a + den_b

    @pl.when(i == nsteps - 1)
    def _fin():
        out_ref[0, 0] = acc_ref[0] / acc_ref[1]


@jax.jit
def kernel(agents, a_f, a_sim, ay, b_f, b_sim):
    C, d = agents.shape
    Na = a_f.shape[0]
    BN = 256
    G = Na // BN
    ay2 = ay.astype(jnp.int32)[:, None]
    out = pl.pallas_call(
        functools.partial(_body, G),
        grid=(G,),
        in_specs=[
            pl.BlockSpec((C, d), lambda i: (0, 0)),
            pl.BlockSpec((BN, d), lambda i: (i, 0)),
            pl.BlockSpec((BN, C), lambda i: (i, 0)),
            pl.BlockSpec((BN, 1), lambda i: (i, 0)),
            pl.BlockSpec((BN, d), lambda i: (i, 0)),
            pl.BlockSpec((BN, C), lambda i: (i, 0)),
        ],
        out_specs=pl.BlockSpec(memory_space=pltpu.SMEM),
        out_shape=jax.ShapeDtypeStruct((1, 1), jnp.float32),
        scratch_shapes=[pltpu.SMEM((2,), jnp.float32)],
    )(agents, a_f, a_sim, ay2, b_f, b_sim)
    return out[0, 0]


# speedup vs baseline: 2.3995x; 1.1352x over previous
import functools

import jax
import jax.numpy as jnp
from jax.experimental import pallas as pl
from jax.experimental.pallas import tpu as pltpu

_MARGIN = 1.0
_SIM_MARGIN = 1.0 - _MARGIN / 2.0


def _terms(f, agents, sim, lab):
    f2 = jnp.sum(f * f, axis=1, keepdims=True)
    a2 = jnp.sum(agents * agents, axis=1)[None, :]
    xdot = jax.lax.dot_general(
        f, agents, (((1,), (1,)), ((), ())), preferred_element_type=jnp.float32
    )
    m1 = (_MARGIN - f2) - a2
    neg = jnp.maximum(0.0, m1 + (xdot + xdot))
    simmask = sim > _SIM_MARGIN
    if lab is not None:
        cols = jax.lax.broadcasted_iota(jnp.int32, sim.shape, 1)
        mask = simmask & (cols != lab)
        sdist = (f2 + a2) - (xdot + xdot)
        pos = jnp.sum(jnp.where(cols == lab, sdist, 0.0))
    else:
        mask = simmask
        pos = 0.0
    cnt = jnp.sum(jnp.where(mask, 1.0, 0.0), axis=1)
    msum = jnp.sum(jnp.where(mask, neg, 0.0), axis=1)
    has = cnt > 0.0
    mean_neg = jnp.where(has, msum / jnp.maximum(cnt, 1.0), 0.0)
    num = pos + jnp.sum(mean_neg)
    den = jnp.sum(jnp.where(has, 1.0, 0.0))
    if lab is not None:
        den = den + float(f.shape[0])
    return num, den


def _body(nsteps, ag_ref, af_ref, asim_ref, ay_ref, bf_ref, bsim_ref,
          out_ref, acc_ref):
    i = pl.program_id(0)

    @pl.when(i == 0)
    def _init():
        acc_ref[0] = 0.0
        acc_ref[1] = 0.0

    agents = ag_ref[...]
    num_a, den_a = _terms(af_ref[...], agents, asim_ref[...], ay_ref[...])
    num_b, den_b = _terms(bf_ref[...], agents, bsim_ref[...], None)
    acc_ref[0] += num_a + num_b
    acc_ref[1] += den_a + den_b

    @pl.when(i == nsteps - 1)
    def _fin():
        out_ref[0, 0] = acc_ref[0] / acc_ref[1]


@jax.jit
def kernel(agents, a_f, a_sim, ay, b_f, b_sim):
    C, d = agents.shape
    Na = a_f.shape[0]
    BN = 512
    G = Na // BN
    ay2 = ay.astype(jnp.int32)[:, None]
    out = pl.pallas_call(
        functools.partial(_body, G),
        grid=(G,),
        in_specs=[
            pl.BlockSpec((C, d), lambda i: (0, 0)),
            pl.BlockSpec((BN, d), lambda i: (i, 0)),
            pl.BlockSpec((BN, C), lambda i: (i, 0)),
            pl.BlockSpec((BN, 1), lambda i: (i, 0)),
            pl.BlockSpec((BN, d), lambda i: (i, 0)),
            pl.BlockSpec((BN, C), lambda i: (i, 0)),
        ],
        out_specs=pl.BlockSpec(memory_space=pltpu.SMEM),
        out_shape=jax.ShapeDtypeStruct((1, 1), jnp.float32),
        scratch_shapes=[pltpu.SMEM((2,), jnp.float32)],
    )(agents, a_f, a_sim, ay2, b_f, b_sim)
    return out[0, 0]
